# Initial kernel scaffold; baseline (speedup 1.0000x reference)
import jax
import jax.numpy as jnp
from jax.experimental import pallas as pl


def kernel(x, edge_index, Wc, bc, W1, b1, W2, b2):
    raise NotImplementedError("write your pallas kernel here")


# speedup vs baseline: 5.7488x; 5.7488x over previous
import functools

import jax
import jax.numpy as jnp
from jax import lax
from jax.experimental import pallas as pl
from jax.experimental.pallas import tpu as pltpu
from jax.experimental.pallas import tpu_sc as plsc

N = 10000
E = 320000
D = 128
DMID = 256

NPAD = 10240
NW = 32
C = 128
K = 79
EPAD = NW * K * C
ROWS_PER_TILE = NPAD // 16

@functools.cache
def _build_sc_agg():
  mesh = plsc.VectorSubcoreMesh(core_axis_name="c", subcore_axis_name="s")

  @functools.partial(
      pl.kernel,
      mesh=mesh,
      out_type=[
          jax.ShapeDtypeStruct((2, NPAD, D), jnp.float32),
          jax.ShapeDtypeStruct((2, NPAD), jnp.float32),
      ],
      scratch_types=[
          pltpu.VMEM((K, C), jnp.int32),
          pltpu.VMEM((K, C), jnp.int32),
          pltpu.VMEM((C, D), jnp.float32),
          pltpu.VMEM((C,), jnp.float32),
          pltpu.VMEM((ROWS_PER_TILE,), jnp.float32),
          pltpu.VMEM_SHARED((NPAD, D), jnp.float32),
          pltpu.VMEM_SHARED((NPAD,), jnp.float32),
          pltpu.SemaphoreType.DMA,
      ],
  )
  def _sc_agg(x_hbm, src_hbm, dst_hbm, agg_hbm, deg_hbm,
              src_v, dst_v, rows_v, ones_v, zdeg_v, agg_sh, deg_sh, sem):
    c = lax.axis_index("c")
    s = lax.axis_index("s")
    w = c * 16 + s
    row0 = s * ROWS_PER_TILE

    def _zrow(t, _):
        r = t // 8
        col = (t % 8) * 16
        rows_v[r, pl.ds(col, 16)] = jnp.zeros((16,), jnp.float32)
        return 0
    lax.fori_loop(0, C * 8, _zrow, 0)

    def _zdeg(t, _):
        zdeg_v[pl.ds(t * 16, 16)] = jnp.zeros((16,), jnp.float32)
        return 0
    lax.fori_loop(0, ROWS_PER_TILE // 16, _zdeg, 0)

    for i in range(8):
        ones_v[pl.ds(i * 16, 16)] = jnp.ones((16,), jnp.float32)

    for kk in range(ROWS_PER_TILE // C):
        pltpu.sync_copy(rows_v, agg_sh.at[pl.ds(row0 + kk * C, C)])
    pltpu.sync_copy(zdeg_v, deg_sh.at[pl.ds(row0, ROWS_PER_TILE)])

    pltpu.sync_copy(src_hbm.at[w], src_v)
    pltpu.sync_copy(dst_hbm.at[w], dst_v)

    plsc.subcore_barrier()

    def _chunk(j, _):
        pltpu.async_copy(x_hbm.at[src_v.at[j]], rows_v, sem).wait()
        pltpu.sync_copy(rows_v, agg_sh.at[dst_v.at[j]], add=True)
        pltpu.sync_copy(ones_v, deg_sh.at[dst_v.at[j]], add=True)
        return 0
    lax.fori_loop(0, K, _chunk, 0)

    plsc.subcore_barrier()

    pltpu.sync_copy(agg_sh.at[pl.ds(row0, ROWS_PER_TILE)],
                    agg_hbm.at[c, pl.ds(row0, ROWS_PER_TILE)])
    pltpu.sync_copy(deg_sh.at[pl.ds(row0, ROWS_PER_TILE)],
                    deg_hbm.at[c, pl.ds(row0, ROWS_PER_TILE)])

  return _sc_agg


BN = 1280


def _tc_body(parts_ref, degc_ref, x_ref, wc_ref, bc_ref, w1_ref, b1_ref,
             w2_ref, b2_ref, out_ref):
    agg = parts_ref[0] + parts_ref[1]
    deg = degc_ref[0] + degc_ref[1]
    agg = agg / jnp.maximum(deg, 1.0)
    conv = jnp.dot(agg, wc_ref[...], preferred_element_type=jnp.float32)
    h = jnp.maximum(conv + bc_ref[...], 0.0)
    z = x_ref[...] + h
    mid = jnp.maximum(
        jnp.dot(z, w1_ref[...], preferred_element_type=jnp.float32) + b1_ref[...], 0.0)
    out_ref[...] = h + jnp.dot(
        mid, w2_ref[...], preferred_element_type=jnp.float32) + b2_ref[...]


_tc_fused = pl.pallas_call(
    _tc_body,
    grid=(NPAD // BN,),
    in_specs=[
        pl.BlockSpec((2, BN, D), lambda i: (0, i, 0)),
        pl.BlockSpec((2, BN, 1), lambda i: (0, i, 0)),
        pl.BlockSpec((BN, D), lambda i: (i, 0)),
        pl.BlockSpec((D, D), lambda i: (0, 0)),
        pl.BlockSpec((1, D), lambda i: (0, 0)),
        pl.BlockSpec((D, DMID), lambda i: (0, 0)),
        pl.BlockSpec((1, DMID), lambda i: (0, 0)),
        pl.BlockSpec((DMID, D), lambda i: (0, 0)),
        pl.BlockSpec((1, D), lambda i: (0, 0)),
    ],
    out_specs=pl.BlockSpec((BN, D), lambda i: (i, 0)),
    out_shape=jax.ShapeDtypeStruct((NPAD, D), jnp.float32),
)


def kernel(x, edge_index, Wc, bc, W1, b1, W2, b2):
    x = x.astype(jnp.float32)
    src = edge_index[0].astype(jnp.int32)
    dst = edge_index[1].astype(jnp.int32)
    pad = EPAD - E
    src_p = jnp.concatenate([src, jnp.zeros((pad,), jnp.int32)]).reshape(NW, K, C)
    dst_p = jnp.concatenate([dst, jnp.full((pad,), NPAD - 1, jnp.int32)]).reshape(NW, K, C)
    x_pad = jnp.zeros((NPAD, D), jnp.float32).at[:N].set(x)
    agg_parts, deg_parts = _build_sc_agg()(x_pad, src_p, dst_p)
    deg_col = deg_parts.reshape(2, NPAD, 1)
    out = _tc_fused(agg_parts, deg_col, x_pad, Wc, bc.reshape(1, D),
                    W1, b1.reshape(1, DMID), W2, b2.reshape(1, D))
    return out[:N]


# speedup vs baseline: 6.0826x; 1.0581x over previous
import functools

import jax
import jax.numpy as jnp
from jax import lax
from jax.experimental import pallas as pl
from jax.experimental.pallas import tpu as pltpu
from jax.experimental.pallas import tpu_sc as plsc

N = 10000
E = 320000
D = 128
DH = 64
DMID = 256

NPAD = 10240
C = 128
K = 160
EPAD = 16 * K * C
ROWS_PER_TILE = NPAD // 16
NBUF = 4


@functools.cache
def _build_sc_agg():
  mesh = plsc.VectorSubcoreMesh(core_axis_name="c", subcore_axis_name="s")

  @functools.partial(
      pl.kernel,
      mesh=mesh,
      out_type=[
          jax.ShapeDtypeStruct((2, NPAD, DH), jnp.float32),
          jax.ShapeDtypeStruct((NPAD,), jnp.float32),
      ],
      scratch_types=[
          pltpu.VMEM((K, C), jnp.int32),
          pltpu.VMEM((K, C), jnp.int32),
          pltpu.VMEM((NBUF, C, DH), jnp.float32),
          pltpu.VMEM((C,), jnp.float32),
          pltpu.VMEM((ROWS_PER_TILE,), jnp.float32),
          pltpu.VMEM_SHARED((NPAD, DH), jnp.float32),
          pltpu.VMEM_SHARED((NPAD,), jnp.float32),
          pltpu.SemaphoreType.DMA,
          pltpu.SemaphoreType.DMA,
          pltpu.SemaphoreType.DMA,
          pltpu.SemaphoreType.DMA,
          pltpu.SemaphoreType.DMA,
          pltpu.SemaphoreType.DMA,
          pltpu.SemaphoreType.DMA,
          pltpu.SemaphoreType.DMA,
          pltpu.SemaphoreType.DMA,
      ],
      compiler_params=pltpu.CompilerParams(use_tc_tiling_on_sc=False),
  )
  def _sc_agg(x2_hbm, src_hbm, dst_hbm, agg_hbm, deg_hbm,
              src_v, dst_v, rows_v, ones_v, zdeg_v, agg_sh, deg_sh,
              g0, g1, g2, g3, s0, s1, s2, s3, dsem):
    gs = (g0, g1, g2, g3)
    ss = (s0, s1, s2, s3)
    c = lax.axis_index("c")
    s = lax.axis_index("s")
    w = c * 16 + s
    row0 = s * ROWS_PER_TILE

    def _zrow(t, _):
        r = t // 4
        col = (t % 4) * 16
        rows_v[0, r, pl.ds(col, 16)] = jnp.zeros((16,), jnp.float32)
        return 0
    lax.fori_loop(0, C * 4, _zrow, 0)

    def _zdeg(t, _):
        zdeg_v[pl.ds(t * 16, 16)] = jnp.zeros((16,), jnp.float32)
        return 0
    lax.fori_loop(0, ROWS_PER_TILE // 16, _zdeg, 0)

    for i in range(8):
        ones_v[pl.ds(i * 16, 16)] = jnp.ones((16,), jnp.float32)

    for kk in range(ROWS_PER_TILE // C):
        pltpu.sync_copy(rows_v.at[0], agg_sh.at[pl.ds(row0 + kk * C, C)])
    pltpu.sync_copy(zdeg_v, deg_sh.at[pl.ds(row0, ROWS_PER_TILE)])

    pltpu.sync_copy(src_hbm.at[w], src_v)
    pltpu.sync_copy(dst_hbm.at[s], dst_v)

    plsc.subcore_barrier()

    def _group(g, _):
        j0 = g * NBUF
        for b in range(NBUF):

            @pl.when(g > 0)
            def _():
                pltpu.make_async_copy(
                    rows_v.at[b], agg_sh.at[dst_v.at[j0 + b]], ss[b]).wait()

            pltpu.async_copy(x2_hbm.at[src_v.at[j0 + b]], rows_v.at[b], gs[b])
        for b in range(NBUF):
            pltpu.make_async_copy(
                x2_hbm.at[src_v.at[j0 + b]], rows_v.at[b], gs[b]).wait()
            pltpu.async_copy(
                rows_v.at[b], agg_sh.at[dst_v.at[j0 + b]], ss[b], add=True)

            @pl.when(c == 0)
            def _():
                pltpu.async_copy(
                    ones_v, deg_sh.at[dst_v.at[j0 + b]], dsem, add=True)
        return 0
    lax.fori_loop(0, K // NBUF, _group, 0)

    for b in range(NBUF):
        pltpu.make_async_copy(
            rows_v.at[b], agg_sh.at[dst_v.at[K - NBUF + b]], ss[b]).wait()

    @pl.when(c == 0)
    def _():
        def _drain(j, _):
            pltpu.make_async_copy(ones_v, deg_sh.at[dst_v.at[j]], dsem).wait()
            return 0
        lax.fori_loop(0, K, _drain, 0)

    plsc.subcore_barrier()

    pltpu.sync_copy(agg_sh.at[pl.ds(row0, ROWS_PER_TILE)],
                    agg_hbm.at[c, pl.ds(row0, ROWS_PER_TILE)])

    @pl.when(c == 0)
    def _():
        pltpu.sync_copy(deg_sh.at[pl.ds(row0, ROWS_PER_TILE)],
                        deg_hbm.at[pl.ds(row0, ROWS_PER_TILE)])

  return _sc_agg


BN = 1280


def _tc_body(parts_ref, degc_ref, x_ref, wc_ref, bc_ref, w1_ref, b1_ref,
             w2_ref, b2_ref, out_ref):
    degm = jnp.maximum(degc_ref[...], 1.0)
    a0 = parts_ref[0] / degm
    a1 = parts_ref[1] / degm
    conv = (jnp.dot(a0, wc_ref[0:DH, :], preferred_element_type=jnp.float32)
            + jnp.dot(a1, wc_ref[DH:D, :], preferred_element_type=jnp.float32))
    h = jnp.maximum(conv + bc_ref[...], 0.0)
    z = x_ref[...] + h
    mid = jnp.maximum(
        jnp.dot(z, w1_ref[...], preferred_element_type=jnp.float32) + b1_ref[...], 0.0)
    out_ref[...] = h + jnp.dot(
        mid, w2_ref[...], preferred_element_type=jnp.float32) + b2_ref[...]


_tc_fused = pl.pallas_call(
    _tc_body,
    grid=(NPAD // BN,),
    in_specs=[
        pl.BlockSpec((2, BN, DH), lambda i: (0, i, 0)),
        pl.BlockSpec((BN, 1), lambda i: (i, 0)),
        pl.BlockSpec((BN, D), lambda i: (i, 0)),
        pl.BlockSpec((D, D), lambda i: (0, 0)),
        pl.BlockSpec((1, D), lambda i: (0, 0)),
        pl.BlockSpec((D, DMID), lambda i: (0, 0)),
        pl.BlockSpec((1, DMID), lambda i: (0, 0)),
        pl.BlockSpec((DMID, D), lambda i: (0, 0)),
        pl.BlockSpec((1, D), lambda i: (0, 0)),
    ],
    out_specs=pl.BlockSpec((BN, D), lambda i: (i, 0)),
    out_shape=jax.ShapeDtypeStruct((NPAD, D), jnp.float32),
)


def kernel(x, edge_index, Wc, bc, W1, b1, W2, b2):
    x = x.astype(jnp.float32)
    src = edge_index[0].astype(jnp.int32)
    dst = edge_index[1].astype(jnp.int32)
    pad = EPAD - E
    src_p = jnp.concatenate([src, jnp.zeros((pad,), jnp.int32)]).reshape(16, K, C)
    dst_p = jnp.concatenate([dst, jnp.full((pad,), NPAD - 1, jnp.int32)]).reshape(16, K, C)
    src2 = jnp.concatenate([src_p, src_p + NPAD])
    x_pad = jnp.zeros((NPAD, D), jnp.float32).at[:N].set(x)
    x2 = jnp.concatenate([x_pad[:, :DH], x_pad[:, DH:]])
    agg_parts, deg = _build_sc_agg()(x2, src2, dst_p)
    out = _tc_fused(agg_parts, deg.reshape(NPAD, 1), x_pad, Wc,
                    bc.reshape(1, D), W1, b1.reshape(1, DMID), W2,
                    b2.reshape(1, D))
    return out[:N]
